# baseline (speedup 1.0000x reference)
import functools

import jax
import jax.numpy as jnp
from jax.experimental import pallas as pl

N_CELL = 50000
N_SPOT = 10000
HIDDEN = 128
OUT_DIM = 64
HEADS = 2
CH = 128


def _layer_norm(x, g, b):
    mu = x.mean(-1, keepdims=True)
    var = ((x - mu) ** 2).mean(-1, keepdims=True)
    return (x - mu) / jnp.sqrt(var + 1e-5) * g + b


def _transformer_conv(p, x_src, x_dst, ei, ea, n_dst):
    src, dst = ei[0], ei[1]
    q = (x_dst @ p["Wq"] + p["bq"]).reshape(-1, HEADS, CH)
    k = (x_src @ p["Wk"] + p["bk"]).reshape(-1, HEADS, CH)
    v = (x_src @ p["Wv"] + p["bv"]).reshape(-1, HEADS, CH)
    e = (ea @ p["We"] + p["be"]).reshape(-1, HEADS, CH)
    kj = k[src] + e
    alpha = (q[dst] * kj).sum(-1) / jnp.sqrt(CH)
    amax = jax.ops.segment_max(alpha, dst, num_segments=n_dst)
    ex = jnp.exp(alpha - amax[dst])
    den = jax.ops.segment_sum(ex, dst, num_segments=n_dst)
    a = ex / jnp.maximum(den[dst], 1e-16)
    msg = (v[src] + e) * a[:, :, None]
    agg = jax.ops.segment_sum(msg, dst, num_segments=n_dst)
    return agg.mean(axis=1) + x_dst @ p["Ws"] + p["bs"]


def _out_proj_kernel(h_ref, w_ref, b_ref, o_ref):
    z = jnp.dot(h_ref[...], w_ref[...], preferred_element_type=jnp.float32)
    z = z + b_ref[...][None, :]
    o_ref[...] = jnp.nan_to_num(z, nan=0.0, posinf=0.0, neginf=0.0)


@functools.partial(jax.jit, static_argnames=("rows",))
def _out_proj(h, w, b, rows):
    n = h.shape[0]
    grid = n // rows
    return pl.pallas_call(
        _out_proj_kernel,
        grid=(grid,),
        in_specs=[
            pl.BlockSpec((rows, HIDDEN), lambda i: (i, 0)),
            pl.BlockSpec((HIDDEN, OUT_DIM), lambda i: (0, 0)),
            pl.BlockSpec((OUT_DIM,), lambda i: (0,)),
        ],
        out_specs=pl.BlockSpec((rows, OUT_DIM), lambda i: (i, 0)),
        out_shape=jax.ShapeDtypeStruct((n, OUT_DIM), jnp.float32),
    )(h, w, b)


def kernel(x_cell, x_spot, edge_index_cs, edge_index_sc, edge_attr_cs, edge_attr_sc, params):
    h_c = jax.nn.relu(x_cell @ params["in_cell"][0] + params["in_cell"][1])
    h_s = jax.nn.relu(x_spot @ params["in_spot"][0] + params["in_spot"][1])
    for lp in params["layers"]:
        new_s = _transformer_conv(lp["cs"], h_c, h_s, edge_index_cs, edge_attr_cs, N_SPOT)
        new_c = _transformer_conv(lp["sc"], h_s, h_c, edge_index_sc, edge_attr_sc, N_CELL)
        h_c = jax.nn.relu(_layer_norm(h_c + new_c, params["ln_cell"][0], params["ln_cell"][1]))
        h_s = jax.nn.relu(_layer_norm(h_s + new_s, params["ln_spot"][0], params["ln_spot"][1]))
    z_c = _out_proj(h_c, params["out_cell"][0], params["out_cell"][1], 400)
    z_s = _out_proj(h_s, params["out_spot"][0], params["out_spot"][1], 400)
    return z_c, z_s


# speedup vs baseline: 5.7282x; 5.7282x over previous
import functools

import jax
import jax.numpy as jnp
from jax import lax
from jax.experimental import pallas as pl
from jax.experimental.pallas import tpu as pltpu
from jax.experimental.pallas import tpu_sc as plsc

N_CELL = 50000
N_SPOT = 10000
HIDDEN = 128
OUT_DIM = 64
E_EDGES = 100000
QT_W = 272

NW = 32
LANES = 16

CFG_SPOT = dict(ndp=10240, nsub=2, seg_cap=2048)
CFG_CELL = dict(ndp=50176, nsub=8, seg_cap=640)

_MESH = dict(core_axis_name="c", subcore_axis_name="s")


def _wid():
    return lax.axis_index("s") * 2 + lax.axis_index("c")


def _iota16():
    return lax.iota(jnp.int32, 16)


def _popcnt(m):
    return jnp.sum(m.astype(jnp.int32))


def _make_partition(cfg):
    ndp, nsub, seg_cap = cfg["ndp"], cfg["nsub"], cfg["seg_cap"]
    r_per_w = ndp // NW
    rs = r_per_w // nsub
    cap = 4096
    nseg = nsub * seg_cap + 16
    ch = 2000
    nchunks = E_EDGES // ch

    out_t = (
        jax.ShapeDtypeStruct((NW, nseg), jnp.int32),
        jax.ShapeDtypeStruct((NW, nseg), jnp.int32),
        jax.ShapeDtypeStruct((NW, nseg), jnp.float32),
        jax.ShapeDtypeStruct((NW, nseg), jnp.float32),
        jax.ShapeDtypeStruct((NW, nseg), jnp.float32),
        jax.ShapeDtypeStruct((NW, nseg), jnp.float32),
    )
    scratch = (
        [pltpu.VMEM((ch,), jnp.int32)] * 2
        + [pltpu.VMEM((ch,), jnp.float32)] * 4
        + [pltpu.VMEM((cap + 16,), jnp.int32)] * 2
        + [pltpu.VMEM((cap + 16,), jnp.float32)] * 4
        + [pltpu.VMEM((nseg,), jnp.int32)] * 2
        + [pltpu.VMEM((nseg,), jnp.float32)] * 4
    )

    @functools.partial(pl.kernel, out_type=out_t,
                       mesh=plsc.VectorSubcoreMesh(**_MESH),
                       compiler_params=pltpu.CompilerParams(
                           needs_layout_passes=False),
                       scratch_types=scratch)
    def part(e_src, e_dst, ea0, ea1, ea2, ea3,
             src_o, dstl_o, e0_o, e1_o, e2_o, e3_o,
             sbuf, dbuf, eb0, eb1, eb2, eb3,
             osrc, odst, oe0, oe1, oe2, oe3,
             fsrc, fdst, fe0, fe1, fe2, fe3):
        w = _wid()
        lo = w * r_per_w
        hi = lo + r_per_w
        it = _iota16()
        ea_in = (ea0, ea1, ea2, ea3)
        ebufs = (eb0, eb1, eb2, eb3)
        oebufs = (oe0, oe1, oe2, oe3)
        febufs = (fe0, fe1, fe2, fe3)

        def init_body(j, _):
            fsrc[pl.ds(j * 16, 16)] = jnp.zeros((16,), jnp.int32)
            fdst[pl.ds(j * 16, 16)] = jnp.full((16,), -1, jnp.int32)
            return 0
        lax.fori_loop(0, nseg // 16, init_body, 0)

        def chunk_body(c, cnt):
            pltpu.sync_copy(e_src.at[pl.ds(c * ch, ch)], sbuf)
            pltpu.sync_copy(e_dst.at[pl.ds(c * ch, ch)], dbuf)
            for m4 in range(4):
                pltpu.sync_copy(ea_in[m4].at[pl.ds(c * ch, ch)], ebufs[m4])

            def vbody(j, cnt):
                d = dbuf[pl.ds(j * 16, 16)]
                s = sbuf[pl.ds(j * 16, 16)]
                m = (d >= lo) & (d < hi)
                tgt = cnt + plsc.cumsum(m.astype(jnp.int32)) - 1
                plsc.store_scatter(osrc, [tgt], s, mask=m)
                plsc.store_scatter(odst, [tgt], d - lo, mask=m)
                for m4 in range(4):
                    ev = ebufs[m4][pl.ds(j * 16, 16)]
                    plsc.store_scatter(oebufs[m4], [tgt], ev, mask=m)
                return jnp.minimum(cnt + _popcnt(m), cap)
            return lax.fori_loop(0, ch // 16, vbody, cnt)
        cnt = lax.fori_loop(0, nchunks, chunk_body, jnp.int32(0))

        def seg_body(s2, _):
            s_lo = s2 * rs
            s_hi = s_lo + rs

            def vbody(j, cs):
                pos = j * 16
                d = odst[pl.ds(pos, 16)]
                sv = osrc[pl.ds(pos, 16)]
                m = (pos + it < cnt) & (d >= s_lo) & (d < s_hi)
                tgt = s2 * seg_cap + cs + plsc.cumsum(m.astype(jnp.int32)) - 1
                plsc.store_scatter(fsrc, [tgt], sv, mask=m)
                plsc.store_scatter(fdst, [tgt], d - s_lo, mask=m)
                for m4 in range(4):
                    ev = oebufs[m4][pl.ds(pos, 16)]
                    plsc.store_scatter(febufs[m4], [tgt], ev, mask=m)
                return jnp.minimum(cs + _popcnt(m), seg_cap - 16)
            lax.fori_loop(0, (cnt + 15) // 16, vbody, jnp.int32(0))
            return 0
        lax.fori_loop(0, nsub, seg_body, 0)

        pltpu.sync_copy(fsrc, src_o.at[w])
        pltpu.sync_copy(fdst, dstl_o.at[w])
        for m4, out_ref in enumerate((e0_o, e1_o, e2_o, e3_o)):
            pltpu.sync_copy(febufs[m4], out_ref.at[w])

    return part


def _make_conv(cfg, n_src):
    ndp, nsub, seg_cap = cfg["ndp"], cfg["nsub"], cfg["seg_cap"]
    r_per_w = ndp // NW
    rs = r_per_w // nsub
    nseg = nsub * seg_cap + 16
    slab_w = rs * QT_W
    nch = seg_cap // 32
    amax_n = ((2 * rs + 15) // 16) * 16

    out_t = jax.ShapeDtypeStruct((ndp * QT_W,), jnp.float32)
    scratch = (
        [pltpu.VMEM((slab_w,), jnp.float32),
         pltpu.VMEM((32, 256), jnp.float32),
         pltpu.VMEM((nseg,), jnp.int32),
         pltpu.VMEM((nseg,), jnp.int32)]
        + [pltpu.VMEM((nseg,), jnp.float32)] * 4
        + [pltpu.VMEM((2 * seg_cap,), jnp.float32),
           pltpu.VMEM((amax_n,), jnp.float32),
           pltpu.SemaphoreType.DMA]
    )

    @functools.partial(pl.kernel, out_type=out_t,
                       mesh=plsc.VectorSubcoreMesh(**_MESH),
                       compiler_params=pltpu.CompilerParams(
                           needs_layout_passes=False),
                       scratch_types=scratch)
    def conv(qt, k, v, src_l, dstl_l, e0_l, e1_l, e2_l, e3_l, agg_o,
             slab, rbuf, srcv, dstv, ev0, ev1, ev2, ev3, albuf, amax, sem):
        w = _wid()
        it = _iota16()
        evs = (ev0, ev1, ev2, ev3)
        pltpu.sync_copy(src_l.at[w], srcv)
        pltpu.sync_copy(dstl_l.at[w], dstv)
        for m4, e_ref in enumerate((e0_l, e1_l, e2_l, e3_l)):
            pltpu.sync_copy(e_ref.at[w], evs[m4])

        def seg_body(s2, _):
            row0 = w * r_per_w + s2 * rs

            pltpu.sync_copy(qt.at[pl.ds(row0 * QT_W, slab_w)], slab)

            def amax_init(j, _):
                amax[pl.ds(j * 16, 16)] = jnp.full((16,), -1e30, jnp.float32)
                return 0
            lax.fori_loop(0, amax_n // 16, amax_init, 0)

            def p1_chunk(c, _):
                base = s2 * seg_cap + c * 32
                d0 = dstv[pl.ds(base, 16)]

                @pl.when(jnp.max(d0) >= 0)
                def _():
                    pltpu.async_copy(k.at[srcv.at[pl.ds(base, 32)]], rbuf,
                                     sem).wait()
                    for g in range(2):
                        pos = base + g * 16
                        dl = dstv[pl.ds(pos, 16)]
                        valid = dl >= 0
                        di = jnp.maximum(dl, 0)
                        diq = di * QT_W
                        rows = it + g * 16

                        def ch_body(chn, accs):
                            a0, a1 = accs
                            cv = jnp.full((16,), 0, jnp.int32) + chn
                            q0 = plsc.load_gather(slab, [diq + chn])
                            k0 = plsc.load_gather(rbuf, [rows, cv])
                            q1 = plsc.load_gather(slab, [diq + (128 + chn)])
                            k1 = plsc.load_gather(rbuf, [rows, cv + 128])
                            return (a0 + q0 * k0, a1 + q1 * k1)
                        a0, a1 = lax.fori_loop(
                            0, 128, ch_body,
                            (jnp.zeros((16,), jnp.float32),
                             jnp.zeros((16,), jnp.float32)))
                        for m in range(4):
                            eam = evs[m][pl.ds(pos, 16)]
                            a0 = a0 + eam * plsc.load_gather(
                                slab, [diq + (256 + m)])
                            a1 = a1 + eam * plsc.load_gather(
                                slab, [diq + (264 + m)])
                        a0 = a0 + plsc.load_gather(slab, [diq + 260])
                        a1 = a1 + plsc.load_gather(slab, [diq + 268])
                        lpos = c * 32 + g * 16
                        albuf[pl.ds(lpos, 16)] = a0
                        albuf[pl.ds(seg_cap + lpos, 16)] = a1

                        for h, av in ((0, a0), (1, a1)):
                            amb = h * rs
                            cur = plsc.load_gather(amax, [amb + di])
                            need = valid & (cur < av)
                            want = jnp.maximum(cur, av)

                            def fix_cond(cr):
                                return jnp.max(cr[0].astype(jnp.int32)) > 0

                            def fix_body(cr):
                                nd, wt = cr
                                plsc.store_scatter(amax, [amb + di], wt,
                                                   mask=nd)
                                cur2 = plsc.load_gather(amax, [amb + di])
                                return (valid & (cur2 < av),
                                        jnp.maximum(cur2, av))
                            lax.while_loop(fix_cond, fix_body, (need, want))
                return 0
            lax.fori_loop(0, nch, p1_chunk, 0)

            def zero_body(j, _):
                slab[pl.ds(j * 16, 16)] = jnp.zeros((16,), jnp.float32)
                return 0
            lax.fori_loop(0, slab_w // 16, zero_body, 0)

            def p2_chunk(c, _):
                base = s2 * seg_cap + c * 32
                d0 = dstv[pl.ds(base, 16)]

                @pl.when(jnp.max(d0) >= 0)
                def _():
                    pltpu.async_copy(v.at[srcv.at[pl.ds(base, 32)]], rbuf,
                                     sem).wait()
                    for g in range(2):
                        pos = base + g * 16
                        dl = dstv[pl.ds(pos, 16)]
                        valid = dl >= 0
                        di = jnp.maximum(dl, 0)
                        diq = di * QT_W
                        rows = it + g * 16
                        lpos = c * 32 + g * 16
                        al0 = albuf[pl.ds(lpos, 16)]
                        al1 = albuf[pl.ds(seg_cap + lpos, 16)]
                        am0 = plsc.load_gather(amax, [di])
                        am1 = plsc.load_gather(amax, [rs + di])
                        z = jnp.zeros((16,), jnp.float32)
                        ex0 = jnp.where(valid, jnp.exp(al0 - am0), z)
                        ex1 = jnp.where(valid, jnp.exp(al1 - am1), z)
                        for m in range(4):
                            eam = evs[m][pl.ds(pos, 16)]
                            plsc.addupdate_scatter(
                                slab, [diq + (256 + m)], ex0 * eam)
                            plsc.addupdate_scatter(
                                slab, [diq + (264 + m)], ex1 * eam)
                        plsc.addupdate_scatter(slab, [diq + 260], ex0)
                        plsc.addupdate_scatter(slab, [diq + 268], ex1)

                        def ch_body(chn, _):
                            cv = jnp.full((16,), 0, jnp.int32) + chn
                            v0 = plsc.load_gather(rbuf, [rows, cv])
                            v1 = plsc.load_gather(rbuf, [rows, cv + 128])
                            plsc.addupdate_scatter(slab, [diq + chn],
                                                   ex0 * v0)
                            plsc.addupdate_scatter(slab, [diq + (128 + chn)],
                                                   ex1 * v1)
                            return 0
                        lax.fori_loop(0, 128, ch_body, 0)
                return 0
            lax.fori_loop(0, nch, p2_chunk, 0)

            pltpu.sync_copy(slab, agg_o.at[pl.ds(row0 * QT_W, slab_w)])
            return 0
        lax.fori_loop(0, nsub, seg_body, 0)

    return conv


_PART_SPOT = _make_partition(CFG_SPOT)
_PART_CELL = _make_partition(CFG_CELL)
_CONV_SPOT = _make_conv(CFG_SPOT, N_CELL)
_CONV_CELL = _make_conv(CFG_CELL, N_SPOT)


_ROWS = 1000


def _inproj_kernel(x_ref, w_ref, b_ref, o_ref):
    z = jnp.dot(x_ref[...], w_ref[...], preferred_element_type=jnp.float32)
    o_ref[...] = jnp.maximum(z + b_ref[...][None, :], 0.0)


def _inproj(x, wmat, b):
    n, din = x.shape
    dout = wmat.shape[1]
    return pl.pallas_call(
        _inproj_kernel,
        grid=(n // _ROWS,),
        in_specs=[
            pl.BlockSpec((_ROWS, din), lambda i: (i, 0)),
            pl.BlockSpec((din, dout), lambda i: (0, 0)),
            pl.BlockSpec((dout,), lambda i: (0,)),
        ],
        out_specs=pl.BlockSpec((_ROWS, dout), lambda i: (i, 0)),
        out_shape=jax.ShapeDtypeStruct((n, dout), jnp.float32),
    )(x, wmat, b)


def _linear_kernel(x_ref, w_ref, b_ref, o_ref):
    z = jnp.dot(x_ref[...], w_ref[...], preferred_element_type=jnp.float32)
    o_ref[...] = z + b_ref[...][None, :]


def _linear(x, wmat, b):
    n, din = x.shape
    dout = wmat.shape[1]
    return pl.pallas_call(
        _linear_kernel,
        grid=(n // _ROWS,),
        in_specs=[
            pl.BlockSpec((_ROWS, din), lambda i: (i, 0)),
            pl.BlockSpec((din, dout), lambda i: (0, 0)),
            pl.BlockSpec((dout,), lambda i: (0,)),
        ],
        out_specs=pl.BlockSpec((_ROWS, dout), lambda i: (i, 0)),
        out_shape=jax.ShapeDtypeStruct((n, dout), jnp.float32),
    )(x, wmat, b)


def _kv_kernel(x_ref, wk_ref, bk_ref, wv_ref, bv_ref, k_ref, v_ref):
    x = x_ref[...]
    k_ref[...] = jnp.dot(x, wk_ref[...],
                         preferred_element_type=jnp.float32) + bk_ref[...][None, :]
    v_ref[...] = jnp.dot(x, wv_ref[...],
                         preferred_element_type=jnp.float32) + bv_ref[...][None, :]


def _kv(x, wk, bk, wv, bv):
    n = x.shape[0]
    return pl.pallas_call(
        _kv_kernel,
        grid=(n // _ROWS,),
        in_specs=[
            pl.BlockSpec((_ROWS, HIDDEN), lambda i: (i, 0)),
            pl.BlockSpec((HIDDEN, 256), lambda i: (0, 0)),
            pl.BlockSpec((256,), lambda i: (0,)),
            pl.BlockSpec((HIDDEN, 256), lambda i: (0, 0)),
            pl.BlockSpec((256,), lambda i: (0,)),
        ],
        out_specs=[
            pl.BlockSpec((_ROWS, 256), lambda i: (i, 0)),
            pl.BlockSpec((_ROWS, 256), lambda i: (i, 0)),
        ],
        out_shape=[
            jax.ShapeDtypeStruct((n, 256), jnp.float32),
            jax.ShapeDtypeStruct((n, 256), jnp.float32),
        ],
    )(x, wk, bk, wv, bv)


def _post_kernel(agg_ref, h_ref, ws_ref, bs_ref, weh_ref, g_ref, b_ref,
                 o_ref):
    agg = agg_ref[...]
    h = h_ref[...]
    weh = weh_ref[...]
    conv = jnp.zeros((agg.shape[0], HIDDEN), jnp.float32)
    for hh in range(2):
        den = agg[:, 256 + hh * 8 + 4]
        inv = jnp.where(den > 0.0, 1.0 / jnp.maximum(den, 1e-30), 0.0)
        acc = agg[:, hh * 128:(hh + 1) * 128]
        for m in range(5):
            acc = acc + agg[:, 256 + hh * 8 + m][:, None] * \
                weh[hh * 8 + m][None, :]
        conv = conv + 0.5 * inv[:, None] * acc
    out = conv + jnp.dot(h, ws_ref[...],
                         preferred_element_type=jnp.float32) + bs_ref[...][None, :]
    r = h + out
    mu = r.mean(-1, keepdims=True)
    var = ((r - mu) ** 2).mean(-1, keepdims=True)
    ln = (r - mu) / jnp.sqrt(var + 1e-5) * g_ref[...][None, :] + \
        b_ref[...][None, :]
    o_ref[...] = jnp.maximum(ln, 0.0)


def _post(agg, h, ws, bs, weh, g, b):
    n = h.shape[0]
    return pl.pallas_call(
        _post_kernel,
        grid=(n // _ROWS,),
        in_specs=[
            pl.BlockSpec((_ROWS, QT_W), lambda i: (i, 0)),
            pl.BlockSpec((_ROWS, HIDDEN), lambda i: (i, 0)),
            pl.BlockSpec((HIDDEN, HIDDEN), lambda i: (0, 0)),
            pl.BlockSpec((HIDDEN,), lambda i: (0,)),
            pl.BlockSpec((16, HIDDEN), lambda i: (0, 0)),
            pl.BlockSpec((HIDDEN,), lambda i: (0,)),
            pl.BlockSpec((HIDDEN,), lambda i: (0,)),
        ],
        out_specs=pl.BlockSpec((_ROWS, HIDDEN), lambda i: (i, 0)),
        out_shape=jax.ShapeDtypeStruct((n, HIDDEN), jnp.float32),
    )(agg, h, ws, bs, weh, g, b)


def _outproj_kernel(h_ref, w_ref, b_ref, o_ref):
    z = jnp.dot(h_ref[...], w_ref[...], preferred_element_type=jnp.float32)
    z = z + b_ref[...][None, :]
    o_ref[...] = jnp.nan_to_num(z, nan=0.0, posinf=0.0, neginf=0.0)


def _outproj(h, wmat, b):
    n = h.shape[0]
    return pl.pallas_call(
        _outproj_kernel,
        grid=(n // _ROWS,),
        in_specs=[
            pl.BlockSpec((_ROWS, HIDDEN), lambda i: (i, 0)),
            pl.BlockSpec((HIDDEN, OUT_DIM), lambda i: (0, 0)),
            pl.BlockSpec((OUT_DIM,), lambda i: (0,)),
        ],
        out_specs=pl.BlockSpec((_ROWS, OUT_DIM), lambda i: (i, 0)),
        out_shape=jax.ShapeDtypeStruct((n, OUT_DIM), jnp.float32),
    )(h, wmat, b)


def _fold_conv_params(p):
    scale = 1.0 / jnp.sqrt(jnp.float32(128.0))
    weh5 = jnp.concatenate(
        [p["We"].reshape(4, 2, 128), p["be"].reshape(1, 2, 128)], 0)
    mh = jnp.transpose(weh5, (1, 2, 0))
    mh = jnp.pad(mh, ((0, 0), (0, 0), (0, 3)))
    mfull = jnp.zeros((2, 128, 2, 8), jnp.float32)
    mfull = mfull.at[0, :, 0].set(mh[0]).at[1, :, 1].set(mh[1])
    mmat = mfull.reshape(256, 16)
    wbig = jnp.concatenate([p["Wq"], p["Wq"] @ mmat], 1) * scale
    bbig = jnp.concatenate([p["bq"], p["bq"] @ mmat]) * scale
    weh = jnp.pad(jnp.transpose(weh5, (1, 0, 2)), ((0, 0), (0, 3), (0, 0)))
    weh = weh.reshape(16, 128)
    return wbig, bbig, weh


def _conv_side(h_src, h_dst, folded, p, part_lists, conv_fn, ndp):
    wbig, bbig, weh = folded
    qt = _linear(h_dst, wbig, bbig)
    qt = jnp.pad(qt, ((0, ndp - qt.shape[0]), (0, 0)))
    k, v = _kv(h_src, p["Wk"], p["bk"], p["Wv"], p["bv"])
    agg = conv_fn(qt.reshape(-1), k, v, *part_lists)
    return agg.reshape(ndp, QT_W)


def kernel(x_cell, x_spot, edge_index_cs, edge_index_sc, edge_attr_cs,
           edge_attr_sc, params):
    h_c = _inproj(x_cell, params["in_cell"][0], params["in_cell"][1])
    h_s = _inproj(x_spot, params["in_spot"][0], params["in_spot"][1])

    part_cs = _PART_SPOT(edge_index_cs[0], edge_index_cs[1],
                         edge_attr_cs[:, 0], edge_attr_cs[:, 1],
                         edge_attr_cs[:, 2], edge_attr_cs[:, 3])
    part_sc = _PART_CELL(edge_index_sc[0], edge_index_sc[1],
                         edge_attr_sc[:, 0], edge_attr_sc[:, 1],
                         edge_attr_sc[:, 2], edge_attr_sc[:, 3])

    for lp in params["layers"]:
        f_cs = _fold_conv_params(lp["cs"])
        f_sc = _fold_conv_params(lp["sc"])
        agg_s = _conv_side(h_c, h_s, f_cs, lp["cs"], part_cs, _CONV_SPOT,
                           CFG_SPOT["ndp"])
        agg_c = _conv_side(h_s, h_c, f_sc, lp["sc"], part_sc, _CONV_CELL,
                           CFG_CELL["ndp"])
        h_s = _post(agg_s[:N_SPOT], h_s, lp["cs"]["Ws"], lp["cs"]["bs"],
                    f_cs[2], params["ln_spot"][0], params["ln_spot"][1])
        h_c = _post(agg_c[:N_CELL], h_c, lp["sc"]["Ws"], lp["sc"]["bs"],
                    f_sc[2], params["ln_cell"][0], params["ln_cell"][1])
    z_c = _outproj(h_c, params["out_cell"][0], params["out_cell"][1])
    z_s = _outproj(h_s, params["out_spot"][0], params["out_spot"][1])
    return z_c, z_s


# speedup vs baseline: 6.5302x; 1.1400x over previous
import functools

import jax
import jax.numpy as jnp
from jax import lax
from jax.experimental import pallas as pl
from jax.experimental.pallas import tpu as pltpu
from jax.experimental.pallas import tpu_sc as plsc

N_CELL = 50000
N_SPOT = 10000
HIDDEN = 128
OUT_DIM = 64
E_EDGES = 100000
QT_W = 272

NW = 32
LANES = 16

CFG_SPOT = dict(ndp=10240, nsub=2, seg_cap=2048)
CFG_CELL = dict(ndp=50176, nsub=8, seg_cap=640)

_MESH = dict(core_axis_name="c", subcore_axis_name="s")


def _wid():
    return lax.axis_index("s") * 2 + lax.axis_index("c")


def _iota16():
    return lax.iota(jnp.int32, 16)


def _popcnt(m):
    return jnp.sum(m.astype(jnp.int32))


def _make_partition(cfg):
    ndp, nsub, seg_cap = cfg["ndp"], cfg["nsub"], cfg["seg_cap"]
    r_per_w = ndp // NW
    rs = r_per_w // nsub
    cap = 4096
    nseg = nsub * seg_cap + 16
    ch = 2000
    nchunks = E_EDGES // ch

    out_t = (
        jax.ShapeDtypeStruct((NW, nseg), jnp.int32),
        jax.ShapeDtypeStruct((NW, nseg), jnp.int32),
        jax.ShapeDtypeStruct((NW, nseg), jnp.float32),
        jax.ShapeDtypeStruct((NW, nseg), jnp.float32),
        jax.ShapeDtypeStruct((NW, nseg), jnp.float32),
        jax.ShapeDtypeStruct((NW, nseg), jnp.float32),
    )
    scratch = (
        [pltpu.VMEM((ch,), jnp.int32)] * 2
        + [pltpu.VMEM((ch,), jnp.float32)] * 4
        + [pltpu.VMEM((cap + 16,), jnp.int32)] * 2
        + [pltpu.VMEM((cap + 16,), jnp.float32)] * 4
        + [pltpu.VMEM((nseg,), jnp.int32)] * 2
        + [pltpu.VMEM((nseg,), jnp.float32)] * 4
        + [pltpu.SemaphoreType.DMA]
    )

    @functools.partial(pl.kernel, out_type=out_t,
                       mesh=plsc.VectorSubcoreMesh(**_MESH),
                       compiler_params=pltpu.CompilerParams(
                           needs_layout_passes=False),
                       scratch_types=scratch)
    def part(e_src, e_dst, ea0, ea1, ea2, ea3,
             src_o, dstl_o, e0_o, e1_o, e2_o, e3_o,
             sbuf, dbuf, eb0, eb1, eb2, eb3,
             osrc, odst, oe0, oe1, oe2, oe3,
             fsrc, fdst, fe0, fe1, fe2, fe3, dsem):
        w = _wid()
        lo = w * r_per_w
        hi = lo + r_per_w
        it = _iota16()
        ea_in = (ea0, ea1, ea2, ea3)
        ebufs = (eb0, eb1, eb2, eb3)
        oebufs = (oe0, oe1, oe2, oe3)
        febufs = (fe0, fe1, fe2, fe3)

        def init_body(j, _):
            fsrc[pl.ds(j * 16, 16)] = jnp.zeros((16,), jnp.int32)
            fdst[pl.ds(j * 16, 16)] = jnp.full((16,), -1, jnp.int32)
            return 0
        lax.fori_loop(0, nseg // 16, init_body, 0)

        def chunk_body(c, cnt):
            descs = [pltpu.async_copy(e_src.at[pl.ds(c * ch, ch)], sbuf, dsem),
                     pltpu.async_copy(e_dst.at[pl.ds(c * ch, ch)], dbuf, dsem)]
            for m4 in range(4):
                descs.append(pltpu.async_copy(ea_in[m4].at[pl.ds(c * ch, ch)],
                                              ebufs[m4], dsem))
            for dd in descs:
                dd.wait()

            def vbody(j, cnt):
                d = dbuf[pl.ds(j * 16, 16)]
                s = sbuf[pl.ds(j * 16, 16)]
                m = (d >= lo) & (d < hi)
                tgt = cnt + plsc.cumsum(m.astype(jnp.int32)) - 1
                plsc.store_scatter(osrc, [tgt], s, mask=m)
                plsc.store_scatter(odst, [tgt], d - lo, mask=m)
                for m4 in range(4):
                    ev = ebufs[m4][pl.ds(j * 16, 16)]
                    plsc.store_scatter(oebufs[m4], [tgt], ev, mask=m)
                return jnp.minimum(cnt + _popcnt(m), cap)
            return lax.fori_loop(0, ch // 16, vbody, cnt)
        cnt = lax.fori_loop(0, nchunks, chunk_body, jnp.int32(0))

        def seg_body(s2, _):
            s_lo = s2 * rs
            s_hi = s_lo + rs

            def vbody(j, cs):
                pos = j * 16
                d = odst[pl.ds(pos, 16)]
                sv = osrc[pl.ds(pos, 16)]
                m = (pos + it < cnt) & (d >= s_lo) & (d < s_hi)
                tgt = s2 * seg_cap + cs + plsc.cumsum(m.astype(jnp.int32)) - 1
                plsc.store_scatter(fsrc, [tgt], sv, mask=m)
                plsc.store_scatter(fdst, [tgt], d - s_lo, mask=m)
                for m4 in range(4):
                    ev = oebufs[m4][pl.ds(pos, 16)]
                    plsc.store_scatter(febufs[m4], [tgt], ev, mask=m)
                return jnp.minimum(cs + _popcnt(m), seg_cap - 16)
            lax.fori_loop(0, (cnt + 15) // 16, vbody, jnp.int32(0))
            return 0
        lax.fori_loop(0, nsub, seg_body, 0)

        pltpu.sync_copy(fsrc, src_o.at[w])
        pltpu.sync_copy(fdst, dstl_o.at[w])
        for m4, out_ref in enumerate((e0_o, e1_o, e2_o, e3_o)):
            pltpu.sync_copy(febufs[m4], out_ref.at[w])

    return part


def _make_conv(cfg, n_src):
    ndp, nsub, seg_cap = cfg["ndp"], cfg["nsub"], cfg["seg_cap"]
    r_per_w = ndp // NW
    rs = r_per_w // nsub
    nseg = nsub * seg_cap + 16
    slab_w = rs * QT_W
    ech = 64
    nch = seg_cap // ech
    amax_n = ((2 * rs + 15) // 16) * 16

    out_t = jax.ShapeDtypeStruct((ndp * QT_W,), jnp.float32)
    scratch = (
        [pltpu.VMEM((slab_w,), jnp.float32),
         pltpu.VMEM((ech, 256), jnp.float32),
         pltpu.VMEM((ech, 256), jnp.float32),
         pltpu.VMEM((nseg,), jnp.int32),
         pltpu.VMEM((nseg,), jnp.int32)]
        + [pltpu.VMEM((nseg,), jnp.float32)] * 4
        + [pltpu.VMEM((2 * seg_cap,), jnp.float32),
           pltpu.VMEM((amax_n,), jnp.float32),
           pltpu.SemaphoreType.DMA,
           pltpu.SemaphoreType.DMA]
    )

    @functools.partial(pl.kernel, out_type=out_t,
                       mesh=plsc.VectorSubcoreMesh(**_MESH),
                       compiler_params=pltpu.CompilerParams(
                           needs_layout_passes=False),
                       scratch_types=scratch)
    def conv(qt, k, v, src_l, dstl_l, e0_l, e1_l, e2_l, e3_l, agg_o,
             slab, rbuf0, rbuf1, srcv, dstv, ev0, ev1, ev2, ev3,
             albuf, amax, sem0, sem1):
        w = _wid()
        it = _iota16()
        evs = (ev0, ev1, ev2, ev3)
        rbufs = (rbuf0, rbuf1)
        sems = (sem0, sem1)
        pltpu.sync_copy(src_l.at[w], srcv)
        pltpu.sync_copy(dstl_l.at[w], dstv)
        for m4, e_ref in enumerate((e0_l, e1_l, e2_l, e3_l)):
            pltpu.sync_copy(e_ref.at[w], evs[m4])

        def p1_compute(base, rbuf):
            for g in range(ech // 16):
                pos = base + g * 16
                dl = dstv[pl.ds(pos, 16)]
                valid = dl >= 0

                @pl.when(jnp.max(dl) >= 0)
                def _():
                    di = jnp.maximum(dl, 0)
                    diq = di * QT_W
                    rows = it + g * 16

                    def ch_body(chn, accs):
                        a0, a1 = accs
                        cv = jnp.full((16,), 0, jnp.int32) + chn
                        q0 = plsc.load_gather(slab, [diq + chn])
                        k0 = plsc.load_gather(rbuf, [rows, cv])
                        q1 = plsc.load_gather(slab, [diq + (128 + chn)])
                        k1 = plsc.load_gather(rbuf, [rows, cv + 128])
                        return (a0 + q0 * k0, a1 + q1 * k1)
                    a0, a1 = lax.fori_loop(
                        0, 128, ch_body,
                        (jnp.zeros((16,), jnp.float32),
                         jnp.zeros((16,), jnp.float32)))
                    for m in range(4):
                        eam = evs[m][pl.ds(pos, 16)]
                        a0 = a0 + eam * plsc.load_gather(
                            slab, [diq + (256 + m)])
                        a1 = a1 + eam * plsc.load_gather(
                            slab, [diq + (264 + m)])
                    a0 = a0 + plsc.load_gather(slab, [diq + 260])
                    a1 = a1 + plsc.load_gather(slab, [diq + 268])
                    lpos = base % seg_cap + g * 16
                    albuf[pl.ds(lpos, 16)] = a0
                    albuf[pl.ds(seg_cap + lpos, 16)] = a1

                    for h, av in ((0, a0), (1, a1)):
                        amb = h * rs
                        cur = plsc.load_gather(amax, [amb + di])
                        need = valid & (cur < av)
                        want = jnp.maximum(cur, av)

                        def fix_cond(cr):
                            return jnp.max(cr[0].astype(jnp.int32)) > 0

                        def fix_body(cr):
                            nd, wt = cr
                            plsc.store_scatter(amax, [amb + di], wt, mask=nd)
                            cur2 = plsc.load_gather(amax, [amb + di])
                            return (valid & (cur2 < av),
                                    jnp.maximum(cur2, av))
                        lax.while_loop(fix_cond, fix_body, (need, want))

        def p2_compute(base, rbuf):
            for g in range(ech // 16):
                pos = base + g * 16
                dl = dstv[pl.ds(pos, 16)]
                valid = dl >= 0

                @pl.when(jnp.max(dl) >= 0)
                def _():
                    di = jnp.maximum(dl, 0)
                    diq = di * QT_W
                    rows = it + g * 16
                    lpos = base % seg_cap + g * 16
                    al0 = albuf[pl.ds(lpos, 16)]
                    al1 = albuf[pl.ds(seg_cap + lpos, 16)]
                    am0 = plsc.load_gather(amax, [di])
                    am1 = plsc.load_gather(amax, [rs + di])
                    z = jnp.zeros((16,), jnp.float32)
                    ex0 = jnp.where(valid, jnp.exp(al0 - am0), z)
                    ex1 = jnp.where(valid, jnp.exp(al1 - am1), z)
                    for m in range(4):
                        eam = evs[m][pl.ds(pos, 16)]
                        plsc.addupdate_scatter(
                            slab, [diq + (256 + m)], ex0 * eam)
                        plsc.addupdate_scatter(
                            slab, [diq + (264 + m)], ex1 * eam)
                    plsc.addupdate_scatter(slab, [diq + 260], ex0)
                    plsc.addupdate_scatter(slab, [diq + 268], ex1)

                    def ch_body(chn, _):
                        cv = jnp.full((16,), 0, jnp.int32) + chn
                        v0 = plsc.load_gather(rbuf, [rows, cv])
                        v1 = plsc.load_gather(rbuf, [rows, cv + 128])
                        plsc.addupdate_scatter(slab, [diq + chn], ex0 * v0)
                        plsc.addupdate_scatter(slab, [diq + (128 + chn)],
                                               ex1 * v1)
                        return 0
                    lax.fori_loop(0, 128, ch_body, 0)

        def pipelined_pass(s2, table, compute):
            def cnt_body(c, n):
                d0 = dstv[pl.ds(s2 * seg_cap + c * ech, 16)]
                return n + jnp.where(jnp.max(d0) >= 0, 1, 0)
            live = lax.fori_loop(0, nch, cnt_body, jnp.int32(0))

            def issue(c, b):
                base = s2 * seg_cap + c * ech
                pltpu.async_copy(table.at[srcv.at[pl.ds(base, ech)]],
                                 rbufs[b], sems[b])

            @pl.when(live > 0)
            def _():
                issue(0, 0)

            def c2_body(c2, _):
                for b in range(2):
                    c = c2 * 2 + b

                    @pl.when(c < live)
                    def _():
                        pltpu.make_async_copy(
                            table.at[srcv.at[pl.ds(0, ech)]],
                            rbufs[b], sems[b]).wait()

                        @pl.when(c + 1 < live)
                        def _():
                            issue(c + 1, 1 - b)
                        compute(s2 * seg_cap + c * ech, rbufs[b])
                return 0
            lax.fori_loop(0, (nch + 1) // 2, c2_body, 0)

        def seg_body(s2, _):
            row0 = w * r_per_w + s2 * rs

            pltpu.sync_copy(qt.at[pl.ds(row0 * QT_W, slab_w)], slab)

            def amax_init(j, _):
                amax[pl.ds(j * 16, 16)] = jnp.full((16,), -1e30, jnp.float32)
                return 0
            lax.fori_loop(0, amax_n // 16, amax_init, 0)

            pipelined_pass(s2, k, p1_compute)

            def zero_body(j, _):
                slab[pl.ds(j * 16, 16)] = jnp.zeros((16,), jnp.float32)
                return 0
            lax.fori_loop(0, slab_w // 16, zero_body, 0)

            pipelined_pass(s2, v, p2_compute)

            pltpu.sync_copy(slab, agg_o.at[pl.ds(row0 * QT_W, slab_w)])
            return 0
        lax.fori_loop(0, nsub, seg_body, 0)

    return conv


_PART_SPOT = _make_partition(CFG_SPOT)
_PART_CELL = _make_partition(CFG_CELL)
_CONV_SPOT = _make_conv(CFG_SPOT, N_CELL)
_CONV_CELL = _make_conv(CFG_CELL, N_SPOT)


_ROWS = 1000


def _inproj_kernel(x_ref, w_ref, b_ref, o_ref):
    z = jnp.dot(x_ref[...], w_ref[...], preferred_element_type=jnp.float32)
    o_ref[...] = jnp.maximum(z + b_ref[...][None, :], 0.0)


def _inproj(x, wmat, b):
    n, din = x.shape
    dout = wmat.shape[1]
    return pl.pallas_call(
        _inproj_kernel,
        grid=(n // _ROWS,),
        in_specs=[
            pl.BlockSpec((_ROWS, din), lambda i: (i, 0)),
            pl.BlockSpec((din, dout), lambda i: (0, 0)),
            pl.BlockSpec((dout,), lambda i: (0,)),
        ],
        out_specs=pl.BlockSpec((_ROWS, dout), lambda i: (i, 0)),
        out_shape=jax.ShapeDtypeStruct((n, dout), jnp.float32),
    )(x, wmat, b)


def _linear_kernel(x_ref, w_ref, b_ref, o_ref):
    z = jnp.dot(x_ref[...], w_ref[...], preferred_element_type=jnp.float32)
    o_ref[...] = z + b_ref[...][None, :]


def _linear(x, wmat, b):
    n, din = x.shape
    dout = wmat.shape[1]
    return pl.pallas_call(
        _linear_kernel,
        grid=(n // _ROWS,),
        in_specs=[
            pl.BlockSpec((_ROWS, din), lambda i: (i, 0)),
            pl.BlockSpec((din, dout), lambda i: (0, 0)),
            pl.BlockSpec((dout,), lambda i: (0,)),
        ],
        out_specs=pl.BlockSpec((_ROWS, dout), lambda i: (i, 0)),
        out_shape=jax.ShapeDtypeStruct((n, dout), jnp.float32),
    )(x, wmat, b)


def _kv_kernel(x_ref, wk_ref, bk_ref, wv_ref, bv_ref, k_ref, v_ref):
    x = x_ref[...]
    k_ref[...] = jnp.dot(x, wk_ref[...],
                         preferred_element_type=jnp.float32) + bk_ref[...][None, :]
    v_ref[...] = jnp.dot(x, wv_ref[...],
                         preferred_element_type=jnp.float32) + bv_ref[...][None, :]


def _kv(x, wk, bk, wv, bv):
    n = x.shape[0]
    return pl.pallas_call(
        _kv_kernel,
        grid=(n // _ROWS,),
        in_specs=[
            pl.BlockSpec((_ROWS, HIDDEN), lambda i: (i, 0)),
            pl.BlockSpec((HIDDEN, 256), lambda i: (0, 0)),
            pl.BlockSpec((256,), lambda i: (0,)),
            pl.BlockSpec((HIDDEN, 256), lambda i: (0, 0)),
            pl.BlockSpec((256,), lambda i: (0,)),
        ],
        out_specs=[
            pl.BlockSpec((_ROWS, 256), lambda i: (i, 0)),
            pl.BlockSpec((_ROWS, 256), lambda i: (i, 0)),
        ],
        out_shape=[
            jax.ShapeDtypeStruct((n, 256), jnp.float32),
            jax.ShapeDtypeStruct((n, 256), jnp.float32),
        ],
    )(x, wk, bk, wv, bv)


def _post_kernel(agg_ref, h_ref, ws_ref, bs_ref, weh_ref, g_ref, b_ref,
                 o_ref):
    agg = agg_ref[...]
    h = h_ref[...]
    weh = weh_ref[...]
    conv = jnp.zeros((agg.shape[0], HIDDEN), jnp.float32)
    for hh in range(2):
        den = agg[:, 256 + hh * 8 + 4]
        inv = jnp.where(den > 0.0, 1.0 / jnp.maximum(den, 1e-30), 0.0)
        acc = agg[:, hh * 128:(hh + 1) * 128]
        for m in range(5):
            acc = acc + agg[:, 256 + hh * 8 + m][:, None] * \
                weh[hh * 8 + m][None, :]
        conv = conv + 0.5 * inv[:, None] * acc
    out = conv + jnp.dot(h, ws_ref[...],
                         preferred_element_type=jnp.float32) + bs_ref[...][None, :]
    r = h + out
    mu = r.mean(-1, keepdims=True)
    var = ((r - mu) ** 2).mean(-1, keepdims=True)
    ln = (r - mu) / jnp.sqrt(var + 1e-5) * g_ref[...][None, :] + \
        b_ref[...][None, :]
    o_ref[...] = jnp.maximum(ln, 0.0)


def _post(agg, h, ws, bs, weh, g, b):
    n = h.shape[0]
    return pl.pallas_call(
        _post_kernel,
        grid=(n // _ROWS,),
        in_specs=[
            pl.BlockSpec((_ROWS, QT_W), lambda i: (i, 0)),
            pl.BlockSpec((_ROWS, HIDDEN), lambda i: (i, 0)),
            pl.BlockSpec((HIDDEN, HIDDEN), lambda i: (0, 0)),
            pl.BlockSpec((HIDDEN,), lambda i: (0,)),
            pl.BlockSpec((16, HIDDEN), lambda i: (0, 0)),
            pl.BlockSpec((HIDDEN,), lambda i: (0,)),
            pl.BlockSpec((HIDDEN,), lambda i: (0,)),
        ],
        out_specs=pl.BlockSpec((_ROWS, HIDDEN), lambda i: (i, 0)),
        out_shape=jax.ShapeDtypeStruct((n, HIDDEN), jnp.float32),
    )(agg, h, ws, bs, weh, g, b)


def _outproj_kernel(h_ref, w_ref, b_ref, o_ref):
    z = jnp.dot(h_ref[...], w_ref[...], preferred_element_type=jnp.float32)
    z = z + b_ref[...][None, :]
    o_ref[...] = jnp.nan_to_num(z, nan=0.0, posinf=0.0, neginf=0.0)


def _outproj(h, wmat, b):
    n = h.shape[0]
    return pl.pallas_call(
        _outproj_kernel,
        grid=(n // _ROWS,),
        in_specs=[
            pl.BlockSpec((_ROWS, HIDDEN), lambda i: (i, 0)),
            pl.BlockSpec((HIDDEN, OUT_DIM), lambda i: (0, 0)),
            pl.BlockSpec((OUT_DIM,), lambda i: (0,)),
        ],
        out_specs=pl.BlockSpec((_ROWS, OUT_DIM), lambda i: (i, 0)),
        out_shape=jax.ShapeDtypeStruct((n, OUT_DIM), jnp.float32),
    )(h, wmat, b)


def _fold_conv_params(p):
    scale = 1.0 / jnp.sqrt(jnp.float32(128.0))
    weh5 = jnp.concatenate(
        [p["We"].reshape(4, 2, 128), p["be"].reshape(1, 2, 128)], 0)
    mh = jnp.transpose(weh5, (1, 2, 0))
    mh = jnp.pad(mh, ((0, 0), (0, 0), (0, 3)))
    mfull = jnp.zeros((2, 128, 2, 8), jnp.float32)
    mfull = mfull.at[0, :, 0].set(mh[0]).at[1, :, 1].set(mh[1])
    mmat = mfull.reshape(256, 16)
    wbig = jnp.concatenate([p["Wq"], p["Wq"] @ mmat], 1) * scale
    bbig = jnp.concatenate([p["bq"], p["bq"] @ mmat]) * scale
    weh = jnp.pad(jnp.transpose(weh5, (1, 0, 2)), ((0, 0), (0, 3), (0, 0)))
    weh = weh.reshape(16, 128)
    return wbig, bbig, weh


def _conv_side(h_src, h_dst, folded, p, part_lists, conv_fn, ndp):
    wbig, bbig, weh = folded
    qt = _linear(h_dst, wbig, bbig)
    qt = jnp.pad(qt, ((0, ndp - qt.shape[0]), (0, 0)))
    k, v = _kv(h_src, p["Wk"], p["bk"], p["Wv"], p["bv"])
    agg = conv_fn(qt.reshape(-1), k, v, *part_lists)
    return agg.reshape(ndp, QT_W)


def kernel(x_cell, x_spot, edge_index_cs, edge_index_sc, edge_attr_cs,
           edge_attr_sc, params):
    h_c = _inproj(x_cell, params["in_cell"][0], params["in_cell"][1])
    h_s = _inproj(x_spot, params["in_spot"][0], params["in_spot"][1])

    part_cs = _PART_SPOT(edge_index_cs[0], edge_index_cs[1],
                         edge_attr_cs[:, 0], edge_attr_cs[:, 1],
                         edge_attr_cs[:, 2], edge_attr_cs[:, 3])
    part_sc = _PART_CELL(edge_index_sc[0], edge_index_sc[1],
                         edge_attr_sc[:, 0], edge_attr_sc[:, 1],
                         edge_attr_sc[:, 2], edge_attr_sc[:, 3])

    for lp in params["layers"]:
        f_cs = _fold_conv_params(lp["cs"])
        f_sc = _fold_conv_params(lp["sc"])
        agg_s = _conv_side(h_c, h_s, f_cs, lp["cs"], part_cs, _CONV_SPOT,
                           CFG_SPOT["ndp"])
        agg_c = _conv_side(h_s, h_c, f_sc, lp["sc"], part_sc, _CONV_CELL,
                           CFG_CELL["ndp"])
        h_s = _post(agg_s[:N_SPOT], h_s, lp["cs"]["Ws"], lp["cs"]["bs"],
                    f_cs[2], params["ln_spot"][0], params["ln_spot"][1])
        h_c = _post(agg_c[:N_CELL], h_c, lp["sc"]["Ws"], lp["sc"]["bs"],
                    f_sc[2], params["ln_cell"][0], params["ln_cell"][1])
    z_c = _outproj(h_c, params["out_cell"][0], params["out_cell"][1])
    z_s = _outproj(h_s, params["out_spot"][0], params["out_spot"][1])
    return z_c, z_s
